# baseline (speedup 1.0000x reference)
import functools

import jax
import jax.numpy as jnp
from jax import lax
from jax.experimental import pallas as pl
from jax.experimental.pallas import tpu as pltpu
from jax.experimental.pallas import tpu_sc as plsc

LANES = 16
PADW = 128
BSTRIPE = 128
NBUF = 2


def _tta_kernel(n_batch, hist, dim, num_cores, num_subcores):
    n_workers = num_cores * num_subcores
    assert n_batch % (BSTRIPE * n_workers) == 0
    n_groups = hist // NBUF
    mesh = plsc.VectorSubcoreMesh(core_axis_name="c", subcore_axis_name="s")

    @functools.partial(
        pl.kernel,
        mesh=mesh,
        out_type=jax.ShapeDtypeStruct((hist, dim, n_batch), jnp.float32),
        compiler_params=pltpu.CompilerParams(needs_layout_passes=False),
        scratch_types=(
            [pltpu.VMEM((1, BSTRIPE), jnp.int32)] * NBUF
            + [pltpu.VMEM((1, dim, BSTRIPE), jnp.float32)] * NBUF
            + [pltpu.VMEM((BSTRIPE, PADW), jnp.float32)] * NBUF
            + [pltpu.SemaphoreType.DMA] * (3 * NBUF)
        ),
    )
    def k(x_hbm, idx_hbm, tab_hbm, out_hbm, *scr):
        idx_v = scr[0:NBUF]
        xb_v = scr[NBUF : 2 * NBUF]
        gb_v = scr[2 * NBUF : 3 * NBUF]
        in_sem = scr[3 * NBUF : 4 * NBUF]
        g_sem = scr[4 * NBUF : 5 * NBUF]
        out_sem = scr[5 * NBUF : 6 * NBUF]

        wid = lax.axis_index("s") * num_cores + lax.axis_index("c")
        b0 = wid * BSTRIPE
        iota16 = lax.iota(jnp.int32, LANES)

        def fire_in(b, h):
            pltpu.async_copy(
                idx_hbm.at[pl.ds(h, 1), pl.ds(b0, BSTRIPE)], idx_v[b], in_sem[b]
            )
            pltpu.async_copy(
                x_hbm.at[pl.ds(h, 1), :, pl.ds(b0, BSTRIPE)], xb_v[b], in_sem[b]
            )

        def wait_in(b):
            pltpu.make_async_copy(
                idx_hbm.at[pl.ds(0, 1), pl.ds(0, BSTRIPE)], idx_v[b], in_sem[b]
            ).wait()
            pltpu.make_async_copy(
                x_hbm.at[pl.ds(0, 1), :, pl.ds(0, BSTRIPE)], xb_v[b], in_sem[b]
            ).wait()

        def wait_out(b):
            pltpu.make_async_copy(
                xb_v[b], out_hbm.at[pl.ds(0, 1), :, pl.ds(0, BSTRIPE)], out_sem[b]
            ).wait()

        for b in range(NBUF):
            fire_in(b, b)

        def group_body(g, carry):
            h0 = g * NBUF
            for b in range(NBUF):
                wait_in(b)
                pltpu.async_copy(tab_hbm.at[idx_v[b].at[0]], gb_v[b], g_sem[b])
            for b in range(NBUF):
                pltpu.make_async_copy(
                    tab_hbm.at[idx_v[b].at[0]], gb_v[b], g_sem[b]
                ).wait()

                def bg_body(bg, carry2, b=b):
                    off = bg * LANES
                    for d in range(dim):
                        vals = plsc.load_gather(gb_v[b], [iota16 + off, jnp.full((LANES,), d, jnp.int32)])
                        plsc.addupdate(
                            xb_v[b].at[0, d, pl.ds(off, LANES)], vals
                        )
                    return carry2

                lax.fori_loop(0, BSTRIPE // LANES, bg_body, 0, unroll=False)
                pltpu.async_copy(
                    xb_v[b],
                    out_hbm.at[pl.ds(h0 + b, 1), :, pl.ds(b0, BSTRIPE)],
                    out_sem[b],
                )
            for b in range(NBUF):
                wait_out(b)

                @pl.when(g < n_groups - 1)
                def _():
                    fire_in(b, h0 + NBUF + b)

            return carry

        lax.fori_loop(0, n_groups, group_body, 0, unroll=False)

    return k


def kernel(x, tta, embedding):
    nb, hist, d = x.shape
    vocab = embedding.shape[0]
    xt = jnp.transpose(x, (1, 2, 0))
    idxt = jnp.transpose((tta.astype(jnp.int32) - 1) % vocab, (1, 0))
    tabp = jnp.pad(embedding, ((0, 0), (0, PADW - d)))
    info = plsc.get_sparse_core_info()
    k = _tta_kernel(nb, hist, d, info.num_cores, info.num_subcores)
    outt = k(xt, idxt, tabp)
    return jnp.transpose(outt, (2, 0, 1))


# speedup vs baseline: 1.4913x; 1.4913x over previous
import functools

import jax
import jax.numpy as jnp
from jax import lax
from jax.experimental import pallas as pl
from jax.experimental.pallas import tpu as pltpu
from jax.experimental.pallas import tpu_sc as plsc

LANES = 16
PADW = 128
BSTRIPE = 128
NBUF = 2


def _tta_kernel(n_batch, hist, dim, num_cores, num_subcores):
    n_workers = num_cores * num_subcores
    assert n_batch % (BSTRIPE * n_workers) == 0
    n_groups = hist // NBUF
    mesh = plsc.VectorSubcoreMesh(core_axis_name="c", subcore_axis_name="s")

    @functools.partial(
        pl.kernel,
        mesh=mesh,
        out_type=jax.ShapeDtypeStruct((hist, dim, n_batch), jnp.float32),
        compiler_params=pltpu.CompilerParams(needs_layout_passes=False),
        scratch_types=(
            [pltpu.VMEM((1, BSTRIPE), jnp.int32)] * NBUF
            + [pltpu.VMEM((1, dim, BSTRIPE), jnp.float32)] * NBUF
            + [pltpu.VMEM((BSTRIPE, PADW), jnp.float32)] * NBUF
            + [pltpu.SemaphoreType.DMA] * (3 * NBUF)
        ),
    )
    def k(x_hbm, idx_hbm, tab_hbm, out_hbm, *scr):
        idx_v = scr[0:NBUF]
        xb_v = scr[NBUF : 2 * NBUF]
        gb_v = scr[2 * NBUF : 3 * NBUF]
        in_sem = scr[3 * NBUF : 4 * NBUF]
        g_sem = scr[4 * NBUF : 5 * NBUF]
        out_sem = scr[5 * NBUF : 6 * NBUF]

        wid = lax.axis_index("s") * num_cores + lax.axis_index("c")
        b0 = wid * BSTRIPE
        iota16 = lax.iota(jnp.int32, LANES)

        def fire_in(b, h):
            pltpu.async_copy(
                idx_hbm.at[pl.ds(h, 1), pl.ds(b0, BSTRIPE)], idx_v[b], in_sem[b]
            )
            pltpu.async_copy(
                x_hbm.at[pl.ds(h, 1), :, pl.ds(b0, BSTRIPE)], xb_v[b], in_sem[b]
            )

        def wait_in(b):
            pltpu.make_async_copy(
                idx_hbm.at[pl.ds(0, 1), pl.ds(0, BSTRIPE)], idx_v[b], in_sem[b]
            ).wait()
            pltpu.make_async_copy(
                x_hbm.at[pl.ds(0, 1), :, pl.ds(0, BSTRIPE)], xb_v[b], in_sem[b]
            ).wait()

        def wait_out(b):
            pltpu.make_async_copy(
                xb_v[b], out_hbm.at[pl.ds(0, 1), :, pl.ds(0, BSTRIPE)], out_sem[b]
            ).wait()

        for b in range(NBUF):
            fire_in(b, b)

        def group_body(g, carry):
            h0 = g * NBUF
            for b in range(NBUF):
                wait_in(b)
                pltpu.async_copy(tab_hbm.at[idx_v[b].at[0]], gb_v[b], g_sem[b])
            for b in range(NBUF):
                pltpu.make_async_copy(
                    tab_hbm.at[idx_v[b].at[0]], gb_v[b], g_sem[b]
                ).wait()

                @plsc.parallel_loop(0, dim, step=1, unroll=4)
                def d_body(d, b=b):
                    didx = jnp.full((LANES,), 0, jnp.int32) + d
                    for bg in range(BSTRIPE // LANES):
                        vals = plsc.load_gather(
                            gb_v[b], [iota16 + bg * LANES, didx]
                        )
                        plsc.addupdate(
                            xb_v[b].at[0, d, pl.ds(bg * LANES, LANES)], vals
                        )
                pltpu.async_copy(
                    xb_v[b],
                    out_hbm.at[pl.ds(h0 + b, 1), :, pl.ds(b0, BSTRIPE)],
                    out_sem[b],
                )
            for b in range(NBUF):
                wait_out(b)

                @pl.when(g < n_groups - 1)
                def _():
                    fire_in(b, h0 + NBUF + b)

            return carry

        lax.fori_loop(0, n_groups, group_body, 0, unroll=False)

    return k


def kernel(x, tta, embedding):
    nb, hist, d = x.shape
    vocab = embedding.shape[0]
    xt = jnp.transpose(x, (1, 2, 0))
    idxt = jnp.transpose((tta.astype(jnp.int32) - 1) % vocab, (1, 0))
    tabp = jnp.pad(embedding, ((0, 0), (0, PADW - d)))
    info = plsc.get_sparse_core_info()
    k = _tta_kernel(nb, hist, d, info.num_cores, info.num_subcores)
    outt = k(xt, idxt, tabp)
    return jnp.transpose(outt, (2, 0, 1))
